# baseline (speedup 1.0000x reference)
import functools

import jax
import jax.numpy as jnp
from jax import lax
from jax.experimental import pallas as pl
from jax.experimental.pallas import tpu as pltpu
from jax.experimental.pallas import tpu_sc as plsc

TEMP = 11.313708498984761
N = 320000
D = 128
NSEG = 10000

ROWS = N // D
TBL = 10016

RPW = 80
TAIL = ROWS - 31 * RPW

TC_BLK = 12800


def _scores_body(q_ref, k_ref, s_ref, gmax_ref):
    i = pl.program_id(0)
    s = jnp.sum(q_ref[...] * k_ref[...], axis=1)
    s2 = s.reshape(TC_BLK // D, D) * (1.0 / TEMP)
    s_ref[...] = s2.reshape(1, TC_BLK // D, D)
    bmax = jnp.broadcast_to(jnp.max(s2, keepdims=True).reshape(1, 1), (1, 16))

    @pl.when(i == 0)
    def _():
        gmax_ref[...] = bmax

    @pl.when(i > 0)
    def _():
        gmax_ref[...] = jnp.maximum(gmax_ref[...], bmax)


def _scores(q, k):
    return pl.pallas_call(
        _scores_body,
        grid=(N // TC_BLK,),
        in_specs=[
            pl.BlockSpec((TC_BLK, D), lambda i: (i, 0)),
            pl.BlockSpec((TC_BLK, D), lambda i: (i, 0)),
        ],
        out_specs=[
            pl.BlockSpec((1, TC_BLK // D, D), lambda i: (i, 0, 0)),
            pl.BlockSpec((1, 16), lambda i: (0, 0)),
        ],
        out_shape=[
            jax.ShapeDtypeStruct((N // TC_BLK, TC_BLK // D, D), jnp.float32),
            jax.ShapeDtypeStruct((1, 16), jnp.float32),
        ],
        compiler_params=pltpu.CompilerParams(
            dimension_semantics=("arbitrary",)),
    )(q, k)


def _sc_segsum_body(s_hbm, idx_hbm, gmax_hbm, zero_hbm, e_hbm, part_hbm,
                    sv, iv, ev, gv, tbl, sem):
    cid = lax.axis_index("c")
    sid = lax.axis_index("s")
    wid = cid * 16 + sid
    base = wid * RPW
    nrows = jnp.where(wid == 31, TAIL, RPW)

    @pl.when(wid < 31)
    def _():
        pltpu.async_copy(s_hbm.at[pl.ds(base, RPW)], sv, sem)
        pltpu.async_copy(idx_hbm.at[pl.ds(base, RPW)], iv, sem)

    @pl.when(wid == 31)
    def _():
        pltpu.async_copy(s_hbm.at[pl.ds(31 * RPW, TAIL)],
                         sv.at[pl.ds(0, TAIL)], sem)
        pltpu.async_copy(idx_hbm.at[pl.ds(31 * RPW, TAIL)],
                         iv.at[pl.ds(0, TAIL)], sem)

    pltpu.sync_copy(gmax_hbm, gv)
    m = gv[0, :]

    @pl.when(wid < 31)
    def _():
        pltpu.make_async_copy(s_hbm.at[pl.ds(base, RPW)], sv, sem).wait()
        pltpu.make_async_copy(idx_hbm.at[pl.ds(base, RPW)], iv, sem).wait()

    @pl.when(wid == 31)
    def _():
        pltpu.make_async_copy(s_hbm.at[pl.ds(31 * RPW, TAIL)],
                              sv.at[pl.ds(0, TAIL)], sem).wait()
        pltpu.make_async_copy(idx_hbm.at[pl.ds(31 * RPW, TAIL)],
                              iv.at[pl.ds(0, TAIL)], sem).wait()

    def exp_row(t, carry):
        for u in range(4):
            r = t * 4 + u
            for c in range(D // 16):
                sl = pl.ds(c * 16, 16)
                ev[r, sl] = jnp.exp(sv[r, sl] - m)
        return carry

    lax.fori_loop(0, nrows // 4, exp_row, 0)

    @pl.when(wid < 31)
    def _():
        pltpu.sync_copy(ev, e_hbm.at[pl.ds(base, RPW)])

    @pl.when(wid == 31)
    def _():
        pltpu.sync_copy(ev.at[pl.ds(0, TAIL)], e_hbm.at[pl.ds(31 * RPW, TAIL)])

    @pl.when(sid == 0)
    def _():
        pltpu.sync_copy(zero_hbm, tbl)

    plsc.subcore_barrier()

    def body(j, carry):
        pltpu.sync_copy(ev.at[j], tbl.at[iv.at[j]], add=True)
        return carry

    lax.fori_loop(0, nrows, body, 0)
    plsc.subcore_barrier()

    @pl.when(sid == 0)
    def _():
        pltpu.sync_copy(tbl, part_hbm.at[cid])


def _sc_normalize_body(e_hbm, idx_hbm, part_hbm, out_hbm, ev, iv, ov, pv, dv,
                       sem):
    cid = lax.axis_index("c")
    sid = lax.axis_index("s")
    wid = cid * 16 + sid
    base = wid * RPW
    nrows = jnp.where(wid == 31, TAIL, RPW)

    @pl.when(wid < 31)
    def _():
        pltpu.async_copy(e_hbm.at[pl.ds(base, RPW)], ev, sem)
        pltpu.async_copy(idx_hbm.at[pl.ds(base, RPW)], iv, sem)

    @pl.when(wid == 31)
    def _():
        pltpu.async_copy(e_hbm.at[pl.ds(31 * RPW, TAIL)],
                         ev.at[pl.ds(0, TAIL)], sem)
        pltpu.async_copy(idx_hbm.at[pl.ds(31 * RPW, TAIL)],
                         iv.at[pl.ds(0, TAIL)], sem)

    pltpu.sync_copy(part_hbm, pv)

    @pl.when(wid < 31)
    def _():
        pltpu.make_async_copy(e_hbm.at[pl.ds(base, RPW)], ev, sem).wait()
        pltpu.make_async_copy(idx_hbm.at[pl.ds(base, RPW)], iv, sem).wait()

    @pl.when(wid == 31)
    def _():
        pltpu.make_async_copy(e_hbm.at[pl.ds(31 * RPW, TAIL)],
                              ev.at[pl.ds(0, TAIL)], sem).wait()
        pltpu.make_async_copy(idx_hbm.at[pl.ds(31 * RPW, TAIL)],
                              iv.at[pl.ds(0, TAIL)], sem).wait()

    def combine(t, carry):
        sl = pl.ds(t * 16, 16)
        dv[sl] = pv[0, sl] + pv[1, sl]
        return carry

    lax.fori_loop(0, TBL // 16, combine, 0)

    def row(t, carry):
        for u in range(4):
            r = t * 4 + u
            for c in range(D // 16):
                sl = pl.ds(c * 16, 16)
                d = plsc.load_gather(dv, [iv[r, sl]])
                ov[r, sl] = ev[r, sl] / d
        return carry

    lax.fori_loop(0, nrows // 4, row, 0)

    @pl.when(wid < 31)
    def _():
        pltpu.sync_copy(ov, out_hbm.at[pl.ds(base, RPW)])

    @pl.when(wid == 31)
    def _():
        pltpu.sync_copy(ov.at[pl.ds(0, TAIL)],
                        out_hbm.at[pl.ds(31 * RPW, TAIL)])


@functools.lru_cache(maxsize=1)
def _sc_kernels():
    mesh = plsc.VectorSubcoreMesh(
        core_axis_name="c", subcore_axis_name="s",
        num_cores=2, num_subcores=16)
    segsum = pl.kernel(
        _sc_segsum_body,
        out_type=(
            jax.ShapeDtypeStruct((ROWS, D), jnp.float32),
            jax.ShapeDtypeStruct((2, TBL), jnp.float32),
        ),
        mesh=mesh,
        compiler_params=pltpu.CompilerParams(needs_layout_passes=False),
        scratch_types=[
            pltpu.VMEM((RPW, D), jnp.float32),
            pltpu.VMEM((RPW, D), jnp.int32),
            pltpu.VMEM((RPW, D), jnp.float32),
            pltpu.VMEM((1, 16), jnp.float32),
            pltpu.VMEM_SHARED((TBL,), jnp.float32),
            pltpu.SemaphoreType.DMA,
        ],
    )
    normalize = pl.kernel(
        _sc_normalize_body,
        out_type=jax.ShapeDtypeStruct((ROWS, D), jnp.float32),
        mesh=mesh,
        compiler_params=pltpu.CompilerParams(needs_layout_passes=False),
        scratch_types=[
            pltpu.VMEM((RPW, D), jnp.float32),
            pltpu.VMEM((RPW, D), jnp.int32),
            pltpu.VMEM((RPW, D), jnp.float32),
            pltpu.VMEM((2, TBL), jnp.float32),
            pltpu.VMEM((TBL,), jnp.float32),
            pltpu.SemaphoreType.DMA,
        ],
    )
    return segsum, normalize


def kernel(q, k, index):
    s, gmax = _scores(q, k)
    zeros = jnp.zeros((TBL,), jnp.float32)
    segsum, normalize = _sc_kernels()
    e, part = segsum(s.reshape(ROWS, D), index.reshape(ROWS, D), gmax, zeros)
    out = normalize(e, index.reshape(ROWS, D), part)
    return out.reshape(-1)


# speedup vs baseline: 1.0092x; 1.0092x over previous
import functools

import jax
import jax.numpy as jnp
from jax import lax
from jax.experimental import pallas as pl
from jax.experimental.pallas import tpu as pltpu
from jax.experimental.pallas import tpu_sc as plsc

TEMP = 11.313708498984761
N = 320000
D = 128
NSEG = 10000

ROWS = N // D
TBL = 10016

RPW = 80
TAIL = ROWS - 31 * RPW

TC_BLK = 12800


def _scores_body(q_ref, k_ref, s_ref, gmax_ref):
    i = pl.program_id(0)
    s = jnp.sum(q_ref[...] * k_ref[...], axis=1)
    s2 = s.reshape(TC_BLK // D, D) * (1.0 / TEMP)
    s_ref[...] = s2.reshape(1, TC_BLK // D, D)
    bmax = jnp.broadcast_to(jnp.max(s2, keepdims=True).reshape(1, 1), (1, 16))

    @pl.when(i == 0)
    def _():
        gmax_ref[...] = bmax

    @pl.when(i > 0)
    def _():
        gmax_ref[...] = jnp.maximum(gmax_ref[...], bmax)


def _scores(q, k):
    return pl.pallas_call(
        _scores_body,
        grid=(N // TC_BLK,),
        in_specs=[
            pl.BlockSpec((TC_BLK, D), lambda i: (i, 0)),
            pl.BlockSpec((TC_BLK, D), lambda i: (i, 0)),
        ],
        out_specs=[
            pl.BlockSpec((1, TC_BLK // D, D), lambda i: (i, 0, 0)),
            pl.BlockSpec((1, 16), lambda i: (0, 0)),
        ],
        out_shape=[
            jax.ShapeDtypeStruct((N // TC_BLK, TC_BLK // D, D), jnp.float32),
            jax.ShapeDtypeStruct((1, 16), jnp.float32),
        ],
        compiler_params=pltpu.CompilerParams(
            dimension_semantics=("arbitrary",)),
    )(q, k)


def _sc_segsum_body(s_hbm, idx_hbm, gmax_hbm, zero_hbm, e_hbm, part_hbm,
                    sv, iv, ev, gv, tbl, sem):
    cid = lax.axis_index("c")
    sid = lax.axis_index("s")
    wid = cid * 16 + sid
    base = wid * RPW
    nrows = jnp.where(wid == 31, TAIL, RPW)

    @pl.when(wid < 31)
    def _():
        pltpu.async_copy(s_hbm.at[pl.ds(base, RPW)], sv, sem)
        pltpu.async_copy(idx_hbm.at[pl.ds(base, RPW)], iv, sem)

    @pl.when(wid == 31)
    def _():
        pltpu.async_copy(s_hbm.at[pl.ds(31 * RPW, TAIL)],
                         sv.at[pl.ds(0, TAIL)], sem)
        pltpu.async_copy(idx_hbm.at[pl.ds(31 * RPW, TAIL)],
                         iv.at[pl.ds(0, TAIL)], sem)

    pltpu.sync_copy(gmax_hbm, gv)
    m = gv[0, :]

    @pl.when(wid < 31)
    def _():
        pltpu.make_async_copy(s_hbm.at[pl.ds(base, RPW)], sv, sem).wait()
        pltpu.make_async_copy(idx_hbm.at[pl.ds(base, RPW)], iv, sem).wait()

    @pl.when(wid == 31)
    def _():
        pltpu.make_async_copy(s_hbm.at[pl.ds(31 * RPW, TAIL)],
                              sv.at[pl.ds(0, TAIL)], sem).wait()
        pltpu.make_async_copy(idx_hbm.at[pl.ds(31 * RPW, TAIL)],
                              iv.at[pl.ds(0, TAIL)], sem).wait()

    def exp_row(r, carry):
        def exp_col(c, carry2):
            sl = pl.ds(c * 16, 16)
            ev[r, sl] = jnp.exp(sv[r, sl] - m)
            return carry2

        lax.fori_loop(0, D // 16, exp_col, 0)
        return carry

    lax.fori_loop(0, nrows, exp_row, 0)

    @pl.when(wid < 31)
    def _():
        pltpu.sync_copy(ev, e_hbm.at[pl.ds(base, RPW)])

    @pl.when(wid == 31)
    def _():
        pltpu.sync_copy(ev.at[pl.ds(0, TAIL)], e_hbm.at[pl.ds(31 * RPW, TAIL)])

    @pl.when(sid == 0)
    def _():
        pltpu.sync_copy(zero_hbm, tbl)

    plsc.subcore_barrier()

    def body(j, carry):
        pltpu.sync_copy(ev.at[j], tbl.at[iv.at[j]], add=True)
        return carry

    lax.fori_loop(0, nrows, body, 0)
    plsc.subcore_barrier()

    @pl.when(sid == 0)
    def _():
        pltpu.sync_copy(tbl, part_hbm.at[cid])


def _sc_normalize_body(e_hbm, idx_hbm, part_hbm, out_hbm, ev, iv, ov, pv, dv,
                       sem):
    cid = lax.axis_index("c")
    sid = lax.axis_index("s")
    wid = cid * 16 + sid
    base = wid * RPW
    nrows = jnp.where(wid == 31, TAIL, RPW)

    @pl.when(wid < 31)
    def _():
        pltpu.async_copy(e_hbm.at[pl.ds(base, RPW)], ev, sem)
        pltpu.async_copy(idx_hbm.at[pl.ds(base, RPW)], iv, sem)

    @pl.when(wid == 31)
    def _():
        pltpu.async_copy(e_hbm.at[pl.ds(31 * RPW, TAIL)],
                         ev.at[pl.ds(0, TAIL)], sem)
        pltpu.async_copy(idx_hbm.at[pl.ds(31 * RPW, TAIL)],
                         iv.at[pl.ds(0, TAIL)], sem)

    pltpu.sync_copy(part_hbm, pv)

    @pl.when(wid < 31)
    def _():
        pltpu.make_async_copy(e_hbm.at[pl.ds(base, RPW)], ev, sem).wait()
        pltpu.make_async_copy(idx_hbm.at[pl.ds(base, RPW)], iv, sem).wait()

    @pl.when(wid == 31)
    def _():
        pltpu.make_async_copy(e_hbm.at[pl.ds(31 * RPW, TAIL)],
                              ev.at[pl.ds(0, TAIL)], sem).wait()
        pltpu.make_async_copy(idx_hbm.at[pl.ds(31 * RPW, TAIL)],
                              iv.at[pl.ds(0, TAIL)], sem).wait()

    def combine(t, carry):
        sl = pl.ds(t * 16, 16)
        dv[sl] = pv[0, sl] + pv[1, sl]
        return carry

    lax.fori_loop(0, TBL // 16, combine, 0)

    def row(r, carry):
        def col(c, carry2):
            sl = pl.ds(c * 16, 16)
            d = plsc.load_gather(dv, [iv[r, sl]])
            ov[r, sl] = ev[r, sl] / d
            return carry2

        lax.fori_loop(0, D // 16, col, 0)
        return carry

    lax.fori_loop(0, nrows, row, 0)

    @pl.when(wid < 31)
    def _():
        pltpu.sync_copy(ov, out_hbm.at[pl.ds(base, RPW)])

    @pl.when(wid == 31)
    def _():
        pltpu.sync_copy(ov.at[pl.ds(0, TAIL)],
                        out_hbm.at[pl.ds(31 * RPW, TAIL)])


@functools.lru_cache(maxsize=1)
def _sc_kernels():
    mesh = plsc.VectorSubcoreMesh(
        core_axis_name="c", subcore_axis_name="s",
        num_cores=2, num_subcores=16)
    segsum = pl.kernel(
        _sc_segsum_body,
        out_type=(
            jax.ShapeDtypeStruct((ROWS, D), jnp.float32),
            jax.ShapeDtypeStruct((2, TBL), jnp.float32),
        ),
        mesh=mesh,
        compiler_params=pltpu.CompilerParams(needs_layout_passes=False),
        scratch_types=[
            pltpu.VMEM((RPW, D), jnp.float32),
            pltpu.VMEM((RPW, D), jnp.int32),
            pltpu.VMEM((RPW, D), jnp.float32),
            pltpu.VMEM((1, 16), jnp.float32),
            pltpu.VMEM_SHARED((TBL,), jnp.float32),
            pltpu.SemaphoreType.DMA,
        ],
    )
    normalize = pl.kernel(
        _sc_normalize_body,
        out_type=jax.ShapeDtypeStruct((ROWS, D), jnp.float32),
        mesh=mesh,
        compiler_params=pltpu.CompilerParams(needs_layout_passes=False),
        scratch_types=[
            pltpu.VMEM((RPW, D), jnp.float32),
            pltpu.VMEM((RPW, D), jnp.int32),
            pltpu.VMEM((RPW, D), jnp.float32),
            pltpu.VMEM((2, TBL), jnp.float32),
            pltpu.VMEM((TBL,), jnp.float32),
            pltpu.SemaphoreType.DMA,
        ],
    )
    return segsum, normalize


def kernel(q, k, index):
    s, gmax = _scores(q, k)
    zeros = jnp.zeros((TBL,), jnp.float32)
    segsum, normalize = _sc_kernels()
    e, part = segsum(s.reshape(ROWS, D), index.reshape(ROWS, D), gmax, zeros)
    out = normalize(e, index.reshape(ROWS, D), part)
    return out.reshape(-1)
